# baseline (speedup 1.0000x reference)
import functools

import jax
import jax.numpy as jnp
from jax import lax
from jax.experimental import pallas as pl
from jax.experimental.pallas import tpu as pltpu
from jax.experimental.pallas import tpu_sc as plsc

N = 10000
E = 320000
D = 128
R = 8

NC = 2
NS = 16
NW = NC * NS
EPT = E // NW
CH = 80
NCHUNK = EPT // CH
RPT = 632
N_PAD = NS * RPT

BN = 1000
GRID = N // BN


def _proj_body(x_ref, w_ref, out_ref):
    x = x_ref[...]
    for r in range(R):
        out_ref[r] = jnp.dot(x, w_ref[r], preferred_element_type=jnp.float32)


def _project(node_feats, W):
    return pl.pallas_call(
        _proj_body,
        grid=(GRID,),
        in_specs=[
            pl.BlockSpec((BN, D), lambda i: (i, 0)),
            pl.BlockSpec((R, D, D), lambda i: (0, 0, 0)),
        ],
        out_specs=pl.BlockSpec((R, BN, D), lambda i: (0, i, 0)),
        out_shape=jax.ShapeDtypeStruct((R, N, D), jnp.float32),
    )(node_feats, W)


def _sc_body(proj_hbm, src_hbm, et_hbm, dst_hbm, zeros_hbm, out_hbm,
             src_v, et_v, dst_v, gidx_v, rows_v, acc_sh, sem):
    cid = lax.axis_index("c")
    sid = lax.axis_index("s")
    wid = sid * NC + cid
    pltpu.sync_copy(zeros_hbm, acc_sh.at[pl.ds(sid * RPT, RPT)])
    plsc.subcore_barrier()

    def chunk(j, carry):
        off = wid * EPT + j * CH
        pltpu.sync_copy(src_hbm.at[pl.ds(off, CH)], src_v)
        pltpu.sync_copy(et_hbm.at[pl.ds(off, CH)], et_v)
        pltpu.sync_copy(dst_hbm.at[pl.ds(off, CH)], dst_v)
        for i in range(CH // 16):
            s = pl.ds(i * 16, 16)
            gidx_v[s] = et_v[s] * N + src_v[s]
        pltpu.async_copy(proj_hbm.at[gidx_v], rows_v, sem).wait()
        pltpu.sync_copy(rows_v, acc_sh.at[dst_v], add=True)
        return carry

    lax.fori_loop(0, NCHUNK, chunk, 0)
    plsc.subcore_barrier()
    pltpu.sync_copy(acc_sh.at[pl.ds(sid * RPT, RPT)],
                    out_hbm.at[cid, pl.ds(sid * RPT, RPT)])


def _scatter(proj_flat, src, et, dst, zeros):
    mesh = plsc.VectorSubcoreMesh(core_axis_name="c", subcore_axis_name="s")
    f = pl.kernel(
        _sc_body,
        out_type=jax.ShapeDtypeStruct((NC, N_PAD, D), jnp.float32),
        mesh=mesh,
        scratch_types=[
            pltpu.VMEM((CH,), jnp.int32),
            pltpu.VMEM((CH,), jnp.int32),
            pltpu.VMEM((CH,), jnp.int32),
            pltpu.VMEM((CH,), jnp.int32),
            pltpu.VMEM((CH, D), jnp.float32),
            pltpu.VMEM_SHARED((N_PAD, D), jnp.float32),
            pltpu.SemaphoreType.DMA,
        ],
    )
    return f(proj_flat, src, et, dst, zeros)


def _fuse_body(part_ref, x_ref, wself_ref, wres_ref, bias_ref, bres_ref,
               new_ref, sums_ref, acc_ref):
    i = pl.program_id(0)
    x = x_ref[...]
    agg = part_ref[0] + part_ref[1]
    selfp = jnp.dot(x, wself_ref[...], preferred_element_type=jnp.float32)
    resp = jnp.dot(x, wres_ref[...], preferred_element_type=jnp.float32)
    h = jnp.maximum(agg + selfp + bias_ref[...], 0.0)
    res = jnp.maximum(resp + bres_ref[...], 0.0)
    new = h + res
    new_ref[...] = new

    @pl.when(i == 0)
    def _():
        acc_ref[...] = jnp.zeros_like(acc_ref)

    acc_ref[0:1, :] += jnp.sum(new, axis=0, keepdims=True)
    acc_ref[1:2, :] += jnp.sum(new * new, axis=0, keepdims=True)

    @pl.when(i == GRID - 1)
    def _():
        sums_ref[...] = acc_ref[...]


def _fuse(part, node_feats, W_self, W_res, bias2, bres2):
    return pl.pallas_call(
        _fuse_body,
        grid=(GRID,),
        in_specs=[
            pl.BlockSpec((NC, BN, D), lambda i: (0, i, 0)),
            pl.BlockSpec((BN, D), lambda i: (i, 0)),
            pl.BlockSpec((D, D), lambda i: (0, 0)),
            pl.BlockSpec((D, D), lambda i: (0, 0)),
            pl.BlockSpec((1, D), lambda i: (0, 0)),
            pl.BlockSpec((1, D), lambda i: (0, 0)),
        ],
        out_specs=[
            pl.BlockSpec((BN, D), lambda i: (i, 0)),
            pl.BlockSpec((2, D), lambda i: (0, 0)),
        ],
        out_shape=[
            jax.ShapeDtypeStruct((N, D), jnp.float32),
            jax.ShapeDtypeStruct((2, D), jnp.float32),
        ],
        scratch_shapes=[pltpu.VMEM((2, D), jnp.float32)],
    )(part, node_feats, W_self, W_res, bias2, bres2)


def _bn_body(new_ref, sums_ref, gamma_ref, beta_ref, out_ref):
    mean = sums_ref[0:1, :] * (1.0 / N)
    var = sums_ref[1:2, :] * (1.0 / N) - mean * mean
    scale = gamma_ref[...] * lax.rsqrt(var + 1e-5)
    out_ref[...] = (new_ref[...] - mean) * scale + beta_ref[...]


def _bn(new, sums, gamma2, beta2):
    return pl.pallas_call(
        _bn_body,
        grid=(GRID,),
        in_specs=[
            pl.BlockSpec((BN, D), lambda i: (i, 0)),
            pl.BlockSpec((2, D), lambda i: (0, 0)),
            pl.BlockSpec((1, D), lambda i: (0, 0)),
            pl.BlockSpec((1, D), lambda i: (0, 0)),
        ],
        out_specs=pl.BlockSpec((BN, D), lambda i: (i, 0)),
        out_shape=jax.ShapeDtypeStruct((N, D), jnp.float32),
    )(new, sums, gamma2, beta2)


def kernel(node_feats, edge_index, etype, W, W_self, bias, W_res, b_res,
           gamma, beta):
    src = edge_index[0]
    dst = edge_index[1]
    proj = _project(node_feats, W).reshape(R * N, D)
    zeros = jnp.zeros((RPT, D), jnp.float32)
    part = _scatter(proj, src, etype, dst, zeros)
    new, sums = _fuse(part, node_feats, W_self, W_res,
                      bias.reshape(1, D), b_res.reshape(1, D))
    return _bn(new, sums, gamma.reshape(1, D), beta.reshape(1, D))


# speedup vs baseline: 1.5498x; 1.5498x over previous
import functools

import jax
import jax.numpy as jnp
from jax import lax
from jax.experimental import pallas as pl
from jax.experimental.pallas import tpu as pltpu
from jax.experimental.pallas import tpu_sc as plsc

N = 10000
E = 320000
D = 128
R = 8

NC = 2
NS = 16
NW = NC * NS
EPT = E // NW
CH = 80
NCHUNK = EPT // CH
RPT = 632
N_PAD = NS * RPT

BN = 1000
GRID = N // BN


def _proj_body(x_ref, w_ref, out_ref):
    x = x_ref[...]
    for r in range(R):
        out_ref[r] = jnp.dot(x, w_ref[r], preferred_element_type=jnp.float32)


def _project(node_feats, W):
    return pl.pallas_call(
        _proj_body,
        grid=(GRID,),
        in_specs=[
            pl.BlockSpec((BN, D), lambda i: (i, 0)),
            pl.BlockSpec((R, D, D), lambda i: (0, 0, 0)),
        ],
        out_specs=pl.BlockSpec((R, BN, D), lambda i: (0, i, 0)),
        out_shape=jax.ShapeDtypeStruct((R, N, D), jnp.float32),
    )(node_feats, W)


def _sc_body(proj_hbm, src_hbm, et_hbm, dst_hbm, zeros_hbm, out_hbm,
             src_v, et_v, dst_v, gidx_v, rows_v, acc_sh, sem, isem):
    cid = lax.axis_index("c")
    sid = lax.axis_index("s")
    wid = sid * NC + cid
    base = wid * EPT
    pltpu.sync_copy(zeros_hbm, acc_sh.at[pl.ds(sid * RPT, RPT)])
    plsc.subcore_barrier()

    def i_start(j, b):
        off = base + j * CH
        pltpu.async_copy(src_hbm.at[pl.ds(off, CH)], src_v.at[b], isem)
        pltpu.async_copy(et_hbm.at[pl.ds(off, CH)], et_v.at[b], isem)
        pltpu.async_copy(dst_hbm.at[pl.ds(off, CH)], dst_v.at[b], isem)

    def i_wait(b):
        pltpu.make_async_copy(src_hbm.at[pl.ds(0, CH)], src_v.at[b],
                              isem).wait()
        pltpu.make_async_copy(et_hbm.at[pl.ds(0, CH)], et_v.at[b],
                              isem).wait()
        pltpu.make_async_copy(dst_hbm.at[pl.ds(0, CH)], dst_v.at[b],
                              isem).wait()

    def do_chunk(j, b):
        i_wait(b)

        @pl.when(j + 1 < NCHUNK)
        def _():
            i_start(j + 1, 1 - b)

        for i in range(CH // 16):
            s = pl.ds(i * 16, 16)
            gidx_v[s] = et_v[b, s] * N + src_v[b, s]
        pltpu.async_copy(proj_hbm.at[gidx_v], rows_v, sem).wait()
        pltpu.sync_copy(rows_v, acc_sh.at[dst_v.at[b]], add=True)

    i_start(0, 0)

    def outer(jo, carry):
        do_chunk(2 * jo, 0)
        do_chunk(2 * jo + 1, 1)
        return carry

    lax.fori_loop(0, NCHUNK // 2, outer, 0)
    do_chunk(NCHUNK - 1, 0)
    plsc.subcore_barrier()
    pltpu.sync_copy(acc_sh.at[pl.ds(sid * RPT, RPT)],
                    out_hbm.at[cid, pl.ds(sid * RPT, RPT)])


def _scatter(proj_flat, src, et, dst, zeros):
    mesh = plsc.VectorSubcoreMesh(core_axis_name="c", subcore_axis_name="s")
    f = pl.kernel(
        _sc_body,
        out_type=jax.ShapeDtypeStruct((NC, N_PAD, D), jnp.float32),
        mesh=mesh,
        scratch_types=[
            pltpu.VMEM((2, CH), jnp.int32),
            pltpu.VMEM((2, CH), jnp.int32),
            pltpu.VMEM((2, CH), jnp.int32),
            pltpu.VMEM((CH,), jnp.int32),
            pltpu.VMEM((CH, D), jnp.float32),
            pltpu.VMEM_SHARED((N_PAD, D), jnp.float32),
            pltpu.SemaphoreType.DMA,
            pltpu.SemaphoreType.DMA,
        ],
    )
    return f(proj_flat, src, et, dst, zeros)


def _fuse_body(part_ref, x_ref, wself_ref, wres_ref, bias_ref, bres_ref,
               new_ref, sums_ref, acc_ref):
    i = pl.program_id(0)
    x = x_ref[...]
    agg = part_ref[0] + part_ref[1]
    selfp = jnp.dot(x, wself_ref[...], preferred_element_type=jnp.float32)
    resp = jnp.dot(x, wres_ref[...], preferred_element_type=jnp.float32)
    h = jnp.maximum(agg + selfp + bias_ref[...], 0.0)
    res = jnp.maximum(resp + bres_ref[...], 0.0)
    new = h + res
    new_ref[...] = new

    @pl.when(i == 0)
    def _():
        acc_ref[...] = jnp.zeros_like(acc_ref)

    acc_ref[0:1, :] += jnp.sum(new, axis=0, keepdims=True)
    acc_ref[1:2, :] += jnp.sum(new * new, axis=0, keepdims=True)

    @pl.when(i == GRID - 1)
    def _():
        sums_ref[...] = acc_ref[...]


def _fuse(part, node_feats, W_self, W_res, bias2, bres2):
    return pl.pallas_call(
        _fuse_body,
        grid=(GRID,),
        in_specs=[
            pl.BlockSpec((NC, BN, D), lambda i: (0, i, 0)),
            pl.BlockSpec((BN, D), lambda i: (i, 0)),
            pl.BlockSpec((D, D), lambda i: (0, 0)),
            pl.BlockSpec((D, D), lambda i: (0, 0)),
            pl.BlockSpec((1, D), lambda i: (0, 0)),
            pl.BlockSpec((1, D), lambda i: (0, 0)),
        ],
        out_specs=[
            pl.BlockSpec((BN, D), lambda i: (i, 0)),
            pl.BlockSpec((2, D), lambda i: (0, 0)),
        ],
        out_shape=[
            jax.ShapeDtypeStruct((N, D), jnp.float32),
            jax.ShapeDtypeStruct((2, D), jnp.float32),
        ],
        scratch_shapes=[pltpu.VMEM((2, D), jnp.float32)],
    )(part, node_feats, W_self, W_res, bias2, bres2)


def _bn_body(new_ref, sums_ref, gamma_ref, beta_ref, out_ref):
    mean = sums_ref[0:1, :] * (1.0 / N)
    var = sums_ref[1:2, :] * (1.0 / N) - mean * mean
    scale = gamma_ref[...] * lax.rsqrt(var + 1e-5)
    out_ref[...] = (new_ref[...] - mean) * scale + beta_ref[...]


def _bn(new, sums, gamma2, beta2):
    return pl.pallas_call(
        _bn_body,
        grid=(GRID,),
        in_specs=[
            pl.BlockSpec((BN, D), lambda i: (i, 0)),
            pl.BlockSpec((2, D), lambda i: (0, 0)),
            pl.BlockSpec((1, D), lambda i: (0, 0)),
            pl.BlockSpec((1, D), lambda i: (0, 0)),
        ],
        out_specs=pl.BlockSpec((BN, D), lambda i: (i, 0)),
        out_shape=jax.ShapeDtypeStruct((N, D), jnp.float32),
    )(new, sums, gamma2, beta2)


def kernel(node_feats, edge_index, etype, W, W_self, bias, W_res, b_res,
           gamma, beta):
    src = edge_index[0]
    dst = edge_index[1]
    proj = _project(node_feats, W).reshape(R * N, D)
    zeros = jnp.zeros((RPT, D), jnp.float32)
    part = _scatter(proj, src, etype, dst, zeros)
    new, sums = _fuse(part, node_feats, W_self, W_res,
                      bias.reshape(1, D), b_res.reshape(1, D))
    return _bn(new, sums, gamma.reshape(1, D), beta.reshape(1, D))
